# baseline (speedup 1.0000x reference)
import functools

import jax
import jax.numpy as jnp
from jax import lax
from jax.experimental import pallas as pl
from jax.experimental.pallas import tpu as pltpu
from jax.experimental.pallas import tpu_sc as plsc

_LANES = 16


@functools.cache
def _build(B, V, D, L):
    NW = 32
    S = B // NW
    HALF = L // 2
    NHF = HALF // _LANES
    REM = HALF - NHF * _LANES
    CH = 800
    NCH = V // CH

    mesh = plsc.VectorSubcoreMesh(core_axis_name="c", subcore_axis_name="s")

    @functools.partial(
        pl.kernel,
        mesh=mesh,
        compiler_params=pltpu.CompilerParams(
            needs_layout_passes=False, use_tc_tiling_on_sc=False),
        out_type=jax.ShapeDtypeStruct((B,), jnp.float32),
        scratch_types=[
            pltpu.VMEM_SHARED((V,), jnp.float32),
            pltpu.VMEM((2, D, CH), jnp.float32),
            pltpu.VMEM((CH,), jnp.float32),
            pltpu.VMEM((S, 2, HALF), jnp.int32),
            pltpu.VMEM((2, 2, HALF), jnp.float32),
            pltpu.VMEM((S,), jnp.float32),
            pltpu.VMEM((_LANES,), jnp.float32),
            pltpu.VMEM((2, _LANES), jnp.float32),
            pltpu.SemaphoreType.DMA,
            pltpu.SemaphoreType.DMA,
            pltpu.SemaphoreType.DMA,
            pltpu.SemaphoreType.DMA,
            pltpu.SemaphoreType.DMA,
        ],
    )
    def pooled(src_hbm, tt_hbm, wb_hbm, out_hbm,
               tw_sh, slab_v, twc_v, idx_v, vals_v, out_v, tw0_v, wb_v,
               psem0, psem1, gsem0, gsem1, isem):
        cid = lax.axis_index("c")
        sid = lax.axis_index("s")
        wid = sid * 2 + cid
        base = wid * S
        psems = (psem0, psem1)
        gsems = (gsem0, gsem1)

        icp = pltpu.make_async_copy(src_hbm.at[pl.ds(base, S)], idx_v, isem)
        icp.start()
        pltpu.sync_copy(wb_hbm, wb_v)

        lanes = lax.iota(jnp.int32, _LANES)
        lane0 = lanes == 0
        zero = jnp.zeros((_LANES,), jnp.float32)
        one = jnp.ones((_LANES,), jnp.float32)

        def allsum(x):
            for sft in (8, 4, 2, 1):
                x = x + jnp.take_along_axis(x, lanes ^ sft, axis=0)
            return x

        wrow = wb_v[0, :]
        ws = [wrow[d] for d in range(D)]

        def p1_copy(i, buf):
            c = sid + _LANES * i
            return pltpu.make_async_copy(
                tt_hbm.at[:, pl.ds(c * CH, CH)], slab_v.at[buf], psems[buf])

        def p1_fire(i, buf):
            @pl.when(sid + _LANES * i < NCH)
            def _():
                p1_copy(i, buf).start()

        def p1_wait(i, buf):
            @pl.when(sid + _LANES * i < NCH)
            def _():
                p1_copy(i, buf).wait()

        def p1_process(i, buf):
            c = sid + _LANES * i

            @pl.when(c < NCH)
            def _():
                def chunk_body(j, _):
                    o = j * _LANES
                    acc = slab_v[buf, 0, pl.ds(o, _LANES)] * ws[0]
                    for d in range(1, D):
                        acc = acc + slab_v[buf, d, pl.ds(o, _LANES)] * ws[d]
                    twc_v[pl.ds(o, _LANES)] = acc
                    return 0

                lax.fori_loop(0, CH // _LANES, chunk_body, 0)
                pltpu.sync_copy(twc_v, tw_sh.at[pl.ds(c * CH, CH)])

        p1_fire(0, 0)

        def p1_body(g, carry):
            i0 = 2 * g
            p1_fire(i0 + 1, 1)
            p1_wait(i0, 0)
            p1_process(i0, 0)
            p1_fire(i0 + 2, 0)
            p1_wait(i0 + 1, 1)
            p1_process(i0 + 1, 1)
            return carry

        lax.fori_loop(0, (NCH // _LANES + 2) // 2, p1_body, 0)

        plsc.subcore_barrier()

        pltpu.sync_copy(tw_sh.at[pl.ds(0, _LANES)], tw0_v)
        tw0 = tw0_v[...][0]
        bs = wb_v[1, :][0]
        l_f = jnp.float32(L)

        icp.wait()

        def g_copy(s, buf, h):
            return pltpu.make_async_copy(
                tw_sh.at[idx_v.at[s, h]],
                vals_v.at[buf, h],
                gsems[buf],
            )

        def g_fire(s, buf):
            for h in range(2):
                g_copy(s, buf, h).start()

        def g_wait(s, buf):
            for h in range(2):
                g_copy(s, buf, h).wait()

        def process(s, buf):
            vsum = zero
            macc = zero
            for h in range(2):
                for k in range(NHF):
                    vsum = vsum + vals_v[buf, h, pl.ds(k * _LANES, _LANES)]
                    chunk = idx_v[s, h, pl.ds(k * _LANES, _LANES)]
                    macc = macc + jnp.where(chunk != 0, one, zero)
                if REM:
                    tailm = lanes >= _LANES - REM
                    tail = vals_v[buf, h, pl.ds(HALF - _LANES, _LANES)]
                    vsum = vsum + jnp.where(tailm, tail, zero)
                    chunk = idx_v[s, h, pl.ds(HALF - _LANES, _LANES)]
                    new = jnp.logical_and(chunk != 0, tailm)
                    macc = macc + jnp.where(new, one, zero)
            len_v = allsum(macc)
            tot = allsum(vsum)
            logit_v = (tot - (l_f - len_v) * tw0) / len_v + bs
            plsc.store_scatter(
                out_v,
                [jnp.broadcast_to(s, (_LANES,)).astype(jnp.int32)],
                logit_v,
                mask=lane0,
            )

        g_fire(0, 0)

        def seq_body(g, carry):
            s0 = 2 * g
            s1 = s0 + 1
            g_fire(s1, 1)
            g_wait(s0, 0)
            process(s0, 0)
            nxt = lax.rem(s0 + 2, S)
            g_fire(nxt, 0)
            g_wait(s1, 1)
            process(s1, 1)
            return carry

        lax.fori_loop(0, S // 2, seq_body, 0)
        g_wait(0, 0)

        for g in range(S // _LANES):
            v = out_v[pl.ds(g * _LANES, _LANES)]
            out_v[pl.ds(g * _LANES, _LANES)] = 1.0 / (1.0 + jnp.exp(-v))

        pltpu.sync_copy(out_v, out_hbm.at[pl.ds(base, S)])

    return pooled


def kernel(src, table, W, b):
    B, L = src.shape
    V, D = table.shape
    src_p = src.reshape(B, 2, L // 2)
    tt = table.T
    wb = jnp.concatenate([
        W.reshape(-1).astype(jnp.float32),
        b.reshape(-1).astype(jnp.float32),
        jnp.zeros((_LANES - 1,), jnp.float32),
    ]).reshape(2, _LANES)
    out = _build(B, V, D, L)(src_p, tt, wb)
    return out.reshape(B, 1)


# speedup vs baseline: 9.3442x; 9.3442x over previous
import functools

import jax
import jax.numpy as jnp
from jax import lax
from jax.experimental import pallas as pl
from jax.experimental.pallas import tpu as pltpu
from jax.experimental.pallas import tpu_sc as plsc

_LANES = 16


@functools.cache
def _build_tw(V, D):
    BLK = 8192
    grid = (V + BLK - 1) // BLK

    def tw_body(w_ref, tt_ref, out_ref):
        out_ref[...] = jnp.sum(tt_ref[...] * w_ref[...].reshape(D, 1), axis=0)

    return pl.pallas_call(
        tw_body,
        grid=(grid,),
        in_specs=[
            pl.BlockSpec((1, D), lambda i: (0, 0)),
            pl.BlockSpec((D, BLK), lambda i: (0, i)),
        ],
        out_specs=pl.BlockSpec((BLK,), lambda i: (i,)),
        out_shape=jax.ShapeDtypeStruct((V,), jnp.float32),
    )


@functools.cache
def _build_pool(B, V, L):
    NW = 32
    S = B // NW
    HALF = L // 2
    NHF = HALF // _LANES
    REM = HALF - NHF * _LANES
    STG = 62496

    mesh = plsc.VectorSubcoreMesh(core_axis_name="c", subcore_axis_name="s")

    @functools.partial(
        pl.kernel,
        mesh=mesh,
        compiler_params=pltpu.CompilerParams(
            needs_layout_passes=False, use_tc_tiling_on_sc=False),
        out_type=jax.ShapeDtypeStruct((B,), jnp.float32),
        scratch_types=[
            pltpu.VMEM_SHARED((V,), jnp.float32),
            pltpu.VMEM((S, 2, HALF), jnp.int32),
            pltpu.VMEM((2, 2, HALF), jnp.float32),
            pltpu.VMEM((S,), jnp.float32),
            pltpu.VMEM((_LANES,), jnp.float32),
            pltpu.VMEM((_LANES,), jnp.float32),
            pltpu.SemaphoreType.DMA,
            pltpu.SemaphoreType.DMA,
            pltpu.SemaphoreType.DMA,
        ],
    )
    def pooled(src_hbm, tw_hbm, bv_hbm, out_hbm,
               tw_sh, idx_v, vals_v, out_v, tw0_v, bv_v,
               gsem0, gsem1, isem):
        cid = lax.axis_index("c")
        sid = lax.axis_index("s")
        wid = sid * 2 + cid
        base = wid * S
        gsems = (gsem0, gsem1)

        icp = pltpu.make_async_copy(src_hbm.at[pl.ds(base, S)], idx_v, isem)
        icp.start()
        pltpu.sync_copy(bv_hbm, bv_v)

        off = sid * STG
        pltpu.sync_copy(tw_hbm.at[pl.ds(off, STG)], tw_sh.at[pl.ds(off, STG)])

        @pl.when(sid == 0)
        def _():
            tail = STG * _LANES
            pltpu.sync_copy(tw_hbm.at[pl.ds(tail, V - STG * _LANES)],
                            tw_sh.at[pl.ds(tail, V - STG * _LANES)])

        plsc.subcore_barrier()

        lanes = lax.iota(jnp.int32, _LANES)
        lane0 = lanes == 0
        zero = jnp.zeros((_LANES,), jnp.float32)
        one = jnp.ones((_LANES,), jnp.float32)

        def allsum(x):
            for sft in (8, 4, 2, 1):
                x = x + jnp.take_along_axis(x, lanes ^ sft, axis=0)
            return x

        pltpu.sync_copy(tw_sh.at[pl.ds(0, _LANES)], tw0_v)
        tw0 = tw0_v[...][0]
        bs = bv_v[...][0]
        l_f = jnp.float32(L)

        icp.wait()

        def g_copy(s, buf, h):
            return pltpu.make_async_copy(
                tw_sh.at[idx_v.at[s, h]],
                vals_v.at[buf, h],
                gsems[buf],
            )

        def g_fire(s, buf):
            for h in range(2):
                g_copy(s, buf, h).start()

        def g_wait(s, buf):
            for h in range(2):
                g_copy(s, buf, h).wait()

        def process(s, buf):
            vsum = zero
            macc = zero
            for h in range(2):
                for k in range(NHF):
                    vsum = vsum + vals_v[buf, h, pl.ds(k * _LANES, _LANES)]
                    chunk = idx_v[s, h, pl.ds(k * _LANES, _LANES)]
                    macc = macc + jnp.where(chunk != 0, one, zero)
                if REM:
                    tailm = lanes >= _LANES - REM
                    tail = vals_v[buf, h, pl.ds(HALF - _LANES, _LANES)]
                    vsum = vsum + jnp.where(tailm, tail, zero)
                    chunk = idx_v[s, h, pl.ds(HALF - _LANES, _LANES)]
                    new = jnp.logical_and(chunk != 0, tailm)
                    macc = macc + jnp.where(new, one, zero)
            len_v = allsum(macc)
            tot = allsum(vsum)
            logit_v = (tot - (l_f - len_v) * tw0) / len_v + bs
            plsc.store_scatter(
                out_v,
                [jnp.broadcast_to(s, (_LANES,)).astype(jnp.int32)],
                logit_v,
                mask=lane0,
            )

        g_fire(0, 0)

        def seq_body(g, carry):
            s0 = 2 * g
            s1 = s0 + 1
            g_fire(s1, 1)
            g_wait(s0, 0)
            process(s0, 0)
            nxt = lax.rem(s0 + 2, S)
            g_fire(nxt, 0)
            g_wait(s1, 1)
            process(s1, 1)
            return carry

        lax.fori_loop(0, S // 2, seq_body, 0)
        g_wait(0, 0)

        for g in range(S // _LANES):
            v = out_v[pl.ds(g * _LANES, _LANES)]
            out_v[pl.ds(g * _LANES, _LANES)] = 1.0 / (1.0 + jnp.exp(-v))

        pltpu.sync_copy(out_v, out_hbm.at[pl.ds(base, S)])

    return pooled


def kernel(src, table, W, b):
    B, L = src.shape
    V, D = table.shape
    src_p = src.reshape(B, 2, L // 2)
    tw = _build_tw(V, D)(W.astype(jnp.float32), table.T)
    bv = jnp.concatenate([
        b.reshape(-1).astype(jnp.float32),
        jnp.zeros((_LANES - 1,), jnp.float32),
    ])
    out = _build_pool(B, V, L)(src_p, tw, bv)
    return out.reshape(B, 1)


# speedup vs baseline: 14.4376x; 1.5451x over previous
import functools

import jax
import jax.numpy as jnp
from jax import lax
from jax.experimental import pallas as pl
from jax.experimental.pallas import tpu as pltpu
from jax.experimental.pallas import tpu_sc as plsc

_LANES = 16


@functools.cache
def _build_tw(V, D):
    BLK = 65536
    grid = (V + BLK - 1) // BLK

    def tw_body(w_ref, tt_ref, out_ref):
        out_ref[...] = jnp.sum(tt_ref[...] * w_ref[...].reshape(D, 1), axis=0)

    return pl.pallas_call(
        tw_body,
        grid=(grid,),
        in_specs=[
            pl.BlockSpec((1, D), lambda i: (0, 0)),
            pl.BlockSpec((D, BLK), lambda i: (0, i)),
        ],
        out_specs=pl.BlockSpec((BLK,), lambda i: (i,)),
        out_shape=jax.ShapeDtypeStruct((V,), jnp.float32),
    )


@functools.cache
def _build_pool(B, V, L):
    NW = 32
    S = B // NW
    HALF = L // 2
    NHF = HALF // _LANES
    REM = HALF - NHF * _LANES
    STG = 62496

    mesh = plsc.VectorSubcoreMesh(core_axis_name="c", subcore_axis_name="s")

    @functools.partial(
        pl.kernel,
        mesh=mesh,
        compiler_params=pltpu.CompilerParams(
            needs_layout_passes=False, use_tc_tiling_on_sc=False),
        out_type=jax.ShapeDtypeStruct((B,), jnp.float32),
        scratch_types=[
            pltpu.VMEM_SHARED((V,), jnp.float32),
            pltpu.VMEM((S, 2, HALF), jnp.int32),
            pltpu.VMEM((2, 2, HALF), jnp.float32),
            pltpu.VMEM((S,), jnp.float32),
            pltpu.VMEM((_LANES,), jnp.float32),
            pltpu.VMEM((_LANES,), jnp.float32),
            pltpu.SemaphoreType.DMA,
            pltpu.SemaphoreType.DMA,
            pltpu.SemaphoreType.DMA,
        ],
    )
    def pooled(src_hbm, tw_hbm, bv_hbm, out_hbm,
               tw_sh, idx_v, vals_v, out_v, tw0_v, bv_v,
               gsem0, gsem1, isem):
        cid = lax.axis_index("c")
        sid = lax.axis_index("s")
        wid = sid * 2 + cid
        base = wid * S
        gsems = (gsem0, gsem1)

        icp = pltpu.make_async_copy(src_hbm.at[pl.ds(base, S)], idx_v, isem)
        icp.start()
        pltpu.sync_copy(bv_hbm, bv_v)

        off = sid * STG
        pltpu.sync_copy(tw_hbm.at[pl.ds(off, STG)], tw_sh.at[pl.ds(off, STG)])

        @pl.when(sid == 0)
        def _():
            tail = STG * _LANES
            pltpu.sync_copy(tw_hbm.at[pl.ds(tail, V - STG * _LANES)],
                            tw_sh.at[pl.ds(tail, V - STG * _LANES)])

        plsc.subcore_barrier()

        lanes = lax.iota(jnp.int32, _LANES)
        lane0 = lanes == 0
        zero = jnp.zeros((_LANES,), jnp.float32)
        one = jnp.ones((_LANES,), jnp.float32)

        def allsum(x):
            for sft in (8, 4, 2, 1):
                x = x + jnp.take_along_axis(x, lanes ^ sft, axis=0)
            return x

        pltpu.sync_copy(tw_sh.at[pl.ds(0, _LANES)], tw0_v)
        tw0 = tw0_v[...][0]
        bs = bv_v[...][0]
        l_f = jnp.float32(L)

        icp.wait()

        def g_copy(s, buf, h):
            return pltpu.make_async_copy(
                tw_sh.at[idx_v.at[s, h]],
                vals_v.at[buf, h],
                gsems[buf],
            )

        def g_fire(s, buf):
            for h in range(2):
                g_copy(s, buf, h).start()

        def g_wait(s, buf):
            for h in range(2):
                g_copy(s, buf, h).wait()

        def process(s, buf):
            vsum = zero
            macc = zero
            for h in range(2):
                for k in range(NHF):
                    vsum = vsum + vals_v[buf, h, pl.ds(k * _LANES, _LANES)]
                    chunk = idx_v[s, h, pl.ds(k * _LANES, _LANES)]
                    macc = macc + jnp.where(chunk != 0, one, zero)
                if REM:
                    tailm = lanes >= _LANES - REM
                    tail = vals_v[buf, h, pl.ds(HALF - _LANES, _LANES)]
                    vsum = vsum + jnp.where(tailm, tail, zero)
                    chunk = idx_v[s, h, pl.ds(HALF - _LANES, _LANES)]
                    new = jnp.logical_and(chunk != 0, tailm)
                    macc = macc + jnp.where(new, one, zero)
            len_v = allsum(macc)
            tot = allsum(vsum)
            logit_v = (tot - (l_f - len_v) * tw0) / len_v + bs
            plsc.store_scatter(
                out_v,
                [jnp.broadcast_to(s, (_LANES,)).astype(jnp.int32)],
                logit_v,
                mask=lane0,
            )

        g_fire(0, 0)

        def seq_body(g, carry):
            s0 = 2 * g
            s1 = s0 + 1
            g_fire(s1, 1)
            g_wait(s0, 0)
            process(s0, 0)
            nxt = lax.rem(s0 + 2, S)
            g_fire(nxt, 0)
            g_wait(s1, 1)
            process(s1, 1)
            return carry

        lax.fori_loop(0, S // 2, seq_body, 0)
        g_wait(0, 0)

        for g in range(S // _LANES):
            v = out_v[pl.ds(g * _LANES, _LANES)]
            out_v[pl.ds(g * _LANES, _LANES)] = 1.0 / (1.0 + jnp.exp(-v))

        pltpu.sync_copy(out_v, out_hbm.at[pl.ds(base, S)])

    return pooled


def kernel(src, table, W, b):
    B, L = src.shape
    V, D = table.shape
    src_p = src.reshape(B, 2, L // 2)
    tw = _build_tw(V, D)(W.astype(jnp.float32), table.T)
    bv = jnp.concatenate([
        b.reshape(-1).astype(jnp.float32),
        jnp.zeros((_LANES - 1,), jnp.float32),
    ])
    out = _build_pool(B, V, L)(src_p, tw, bv)
    return out.reshape(B, 1)


# speedup vs baseline: 14.9774x; 1.0374x over previous
import functools

import jax
import jax.numpy as jnp
from jax import lax
from jax.experimental import pallas as pl
from jax.experimental.pallas import tpu as pltpu
from jax.experimental.pallas import tpu_sc as plsc

_LANES = 16


@functools.cache
def _build_tw(V, D):
    BLK = 131072
    grid = (V + BLK - 1) // BLK

    def tw_body(w_ref, tt_ref, out_ref):
        out_ref[...] = jnp.sum(tt_ref[...] * w_ref[...].reshape(D, 1), axis=0)

    return pl.pallas_call(
        tw_body,
        grid=(grid,),
        in_specs=[
            pl.BlockSpec((1, D), lambda i: (0, 0)),
            pl.BlockSpec((D, BLK), lambda i: (0, i)),
        ],
        out_specs=pl.BlockSpec((BLK,), lambda i: (i,)),
        out_shape=jax.ShapeDtypeStruct((V,), jnp.float32),
    )


@functools.cache
def _build_pool(B, V, L):
    NW = 32
    S = B // NW
    HALF = L // 2
    NHF = HALF // _LANES
    REM = HALF - NHF * _LANES
    STG = 62496

    mesh = plsc.VectorSubcoreMesh(core_axis_name="c", subcore_axis_name="s")

    @functools.partial(
        pl.kernel,
        mesh=mesh,
        compiler_params=pltpu.CompilerParams(
            needs_layout_passes=False, use_tc_tiling_on_sc=False),
        out_type=jax.ShapeDtypeStruct((B,), jnp.float32),
        scratch_types=[
            pltpu.VMEM_SHARED((V,), jnp.float32),
            pltpu.VMEM((S, 2, HALF), jnp.int32),
            pltpu.VMEM((2, 2, HALF), jnp.float32),
            pltpu.VMEM((S,), jnp.float32),
            pltpu.VMEM((_LANES,), jnp.float32),
            pltpu.VMEM((_LANES,), jnp.float32),
            pltpu.SemaphoreType.DMA,
            pltpu.SemaphoreType.DMA,
            pltpu.SemaphoreType.DMA,
        ],
    )
    def pooled(src_hbm, tw_hbm, bv_hbm, out_hbm,
               tw_sh, idx_v, vals_v, out_v, tw0_v, bv_v,
               gsem0, gsem1, isem):
        cid = lax.axis_index("c")
        sid = lax.axis_index("s")
        wid = sid * 2 + cid
        base = wid * S
        gsems = (gsem0, gsem1)

        icp = pltpu.make_async_copy(src_hbm.at[pl.ds(base, S)], idx_v, isem)
        icp.start()
        pltpu.sync_copy(bv_hbm, bv_v)

        off = sid * STG
        pltpu.sync_copy(tw_hbm.at[pl.ds(off, STG)], tw_sh.at[pl.ds(off, STG)])

        @pl.when(sid == 0)
        def _():
            tail = STG * _LANES
            pltpu.sync_copy(tw_hbm.at[pl.ds(tail, V - STG * _LANES)],
                            tw_sh.at[pl.ds(tail, V - STG * _LANES)])

        plsc.subcore_barrier()

        lanes = lax.iota(jnp.int32, _LANES)
        lane0 = lanes == 0
        zero = jnp.zeros((_LANES,), jnp.float32)
        one = jnp.ones((_LANES,), jnp.float32)

        def allsum(x):
            for sft in (8, 4, 2, 1):
                x = x + jnp.take_along_axis(x, lanes ^ sft, axis=0)
            return x

        pltpu.sync_copy(tw_sh.at[pl.ds(0, _LANES)], tw0_v)
        tw0 = tw0_v[...][0]
        bs = bv_v[...][0]
        l_f = jnp.float32(L)

        icp.wait()

        def g_copy(s, buf, h):
            return pltpu.make_async_copy(
                tw_sh.at[idx_v.at[s, h]],
                vals_v.at[buf, h],
                gsems[buf],
            )

        def g_fire(s, buf):
            for h in range(2):
                g_copy(s, buf, h).start()

        def g_wait(s, buf):
            for h in range(2):
                g_copy(s, buf, h).wait()

        def process(s, buf):
            vsum = zero
            macc = zero
            for h in range(2):
                for k in range(NHF):
                    vsum = vsum + vals_v[buf, h, pl.ds(k * _LANES, _LANES)]
                    chunk = idx_v[s, h, pl.ds(k * _LANES, _LANES)]
                    macc = macc + jnp.where(chunk != 0, one, zero)
                if REM:
                    tailm = lanes >= _LANES - REM
                    tail = vals_v[buf, h, pl.ds(HALF - _LANES, _LANES)]
                    vsum = vsum + jnp.where(tailm, tail, zero)
                    chunk = idx_v[s, h, pl.ds(HALF - _LANES, _LANES)]
                    new = jnp.logical_and(chunk != 0, tailm)
                    macc = macc + jnp.where(new, one, zero)
            len_v = allsum(macc)
            tot = allsum(vsum)
            logit_v = (tot - (l_f - len_v) * tw0) / len_v + bs
            plsc.store_scatter(
                out_v,
                [jnp.broadcast_to(s, (_LANES,)).astype(jnp.int32)],
                logit_v,
                mask=lane0,
            )

        g_fire(0, 0)

        def seq_body(g, carry):
            s0 = 2 * g
            s1 = s0 + 1
            g_fire(s1, 1)
            g_wait(s0, 0)
            process(s0, 0)
            nxt = lax.rem(s0 + 2, S)
            g_fire(nxt, 0)
            g_wait(s1, 1)
            process(s1, 1)
            return carry

        lax.fori_loop(0, S // 2, seq_body, 0)
        g_wait(0, 0)

        for g in range(S // _LANES):
            v = out_v[pl.ds(g * _LANES, _LANES)]
            out_v[pl.ds(g * _LANES, _LANES)] = 1.0 / (1.0 + jnp.exp(-v))

        pltpu.sync_copy(out_v, out_hbm.at[pl.ds(base, S)])

    return pooled


def kernel(src, table, W, b):
    B, L = src.shape
    V, D = table.shape
    src_p = src.reshape(B, 2, L // 2)
    tw = _build_tw(V, D)(W.astype(jnp.float32), table.T)
    bv = jnp.concatenate([
        b.reshape(-1).astype(jnp.float32),
        jnp.zeros((_LANES - 1,), jnp.float32),
    ])
    out = _build_pool(B, V, L)(src_p, tw, bv)
    return out.reshape(B, 1)
